# baseline (speedup 1.0000x reference)
import jax
import jax.numpy as jnp
from jax import lax
from jax.experimental import pallas as pl
from jax.experimental.pallas import tpu as pltpu
from jax.experimental.pallas import tpu_sc as plsc

VOCAB = 1000000
DIM = 64
BATCH = 4096
HIST = 200

NC = 2
NS = 16
NW = NC * NS
TW = VOCAB // 2
BB = BATCH // NW
ROWS_BYTES = BB * 128 * 4
SLAB_BYTES = DIM * BB * 4


def _emb_body(xT, tablew, out_hbm, idx_all, widx0, widx1, hbuf0, hbuf1,
              rows0, rows1, obuf0, obuf1, g0, g1, o0, o1):
    w = lax.axis_index("s") * NC + lax.axis_index("c")
    col0 = w * BB

    pltpu.sync_copy(xT.at[:, pl.ds(col0, BB)], idx_all)

    jconst = [lax.iota(jnp.int32, 16) + (16 * g) for g in range(8)]

    def stage_a(t, widx, hbuf, rows, gsem):
        for g in range(8):
            v = idx_all[t, pl.ds(16 * g, 16)]
            widx[pl.ds(16 * g, 16)] = lax.shift_right_logical(v, 1)
            hbuf[pl.ds(16 * g, 16)] = lax.shift_left(lax.bitwise_and(v, 1), 6)
        pltpu.async_copy(tablew.at[widx], rows, gsem)

    def wait_gather(rows, gsem):
        pltpu.make_async_copy(tablew.at[pl.ds(0, BB)], rows, gsem).wait()

    def extract(rows, obuf, hbuf):
        hvecs = [hbuf[pl.ds(16 * g, 16)] for g in range(8)]
        for d in range(DIM):
            for g in range(8):
                vec = plsc.load_gather(rows, [jconst[g], hvecs[g] + d])
                obuf[d, pl.ds(16 * g, 16)] = vec

    def fire_wb(t, obuf, osem):
        pltpu.async_copy(obuf, out_hbm.at[t, :, pl.ds(col0, BB)], osem)

    def wait_wb(obuf, osem):
        pltpu.make_async_copy(obuf, out_hbm.at[0, :, pl.ds(col0, BB)],
                              osem).wait()

    stage_a(0, widx0, hbuf0, rows0, g0)
    stage_a(1, widx1, hbuf1, rows1, g1)

    def step(k, carry):
        s0 = 2 * k

        wait_gather(rows0, g0)

        @pl.when(s0 >= 2)
        def _():
            wait_wb(obuf0, o0)

        extract(rows0, obuf0, hbuf0)
        fire_wb(s0, obuf0, o0)

        @pl.when(s0 + 2 < HIST)
        def _():
            stage_a(s0 + 2, widx0, hbuf0, rows0, g0)

        wait_gather(rows1, g1)

        @pl.when(s0 + 1 >= 2)
        def _():
            wait_wb(obuf1, o1)

        extract(rows1, obuf1, hbuf1)
        fire_wb(s0 + 1, obuf1, o1)

        @pl.when(s0 + 3 < HIST)
        def _():
            stage_a(s0 + 3, widx1, hbuf1, rows1, g1)

        return carry

    lax.fori_loop(0, HIST // 2, step, 0)
    wait_wb(obuf0, o0)
    wait_wb(obuf1, o1)


@jax.jit
def _emb(xT, tablew):
    mesh = plsc.VectorSubcoreMesh(core_axis_name="c", subcore_axis_name="s")
    run = pl.kernel(
        _emb_body,
        out_type=jax.ShapeDtypeStruct((HIST, DIM, BATCH), jnp.float32),
        mesh=mesh,
        scratch_types=[
            pltpu.VMEM((HIST, BB), jnp.int32),
            pltpu.VMEM((BB,), jnp.int32),
            pltpu.VMEM((BB,), jnp.int32),
            pltpu.VMEM((BB,), jnp.int32),
            pltpu.VMEM((BB,), jnp.int32),
            pltpu.VMEM((BB, 128), jnp.float32),
            pltpu.VMEM((BB, 128), jnp.float32),
            pltpu.VMEM((DIM, BB), jnp.float32),
            pltpu.VMEM((DIM, BB), jnp.float32),
            pltpu.SemaphoreType.DMA,
            pltpu.SemaphoreType.DMA,
            pltpu.SemaphoreType.DMA,
            pltpu.SemaphoreType.DMA,
        ],
        compiler_params=pltpu.CompilerParams(needs_layout_passes=False),
    )
    return run(xT, tablew)


def kernel(x, table):
    xT = x.T.astype(jnp.int32)
    tablew = table.reshape(TW, 128)
    outT = _emb(xT, tablew)
    return outT.transpose(2, 0, 1)


# speedup vs baseline: 1.9210x; 1.9210x over previous
import jax
import jax.numpy as jnp
from jax import lax
from jax.experimental import pallas as pl
from jax.experimental.pallas import tpu as pltpu
from jax.experimental.pallas import tpu_sc as plsc

VOCAB = 1000000
DIM = 64
BATCH = 4096
HIST = 200

B = BATCH * HIST
NC = 2
NS = 16
NW = NC * NS
BPW = B // NW
G = 128
NU = BPW // G
NBUF = 6
AHEAD = 3
ROUNDS = (NU - 2) // NBUF


def _emb_body(xb, tfat, out2, idx_all, *bufs_and_sems):
    rows = bufs_and_sems[:NBUF]
    gsem = bufs_and_sems[NBUF:2 * NBUF]
    osem = bufs_and_sems[2 * NBUF:3 * NBUF]

    w = lax.axis_index("s") * NC + lax.axis_index("c")
    base = w * BPW

    pltpu.sync_copy(xb.at[w], idx_all)

    def fire_gather(u, j):
        pltpu.async_copy(tfat.at[idx_all.at[u]], rows[j], gsem[j])

    def drain_gather(j):
        pltpu.make_async_copy(tfat.at[pl.ds(0, G)], rows[j], gsem[j]).wait()

    def fire_wb(u, j):
        pltpu.async_copy(rows[j], out2.at[pl.ds(base + u * G, G)], osem[j])

    def drain_wb(j):
        pltpu.make_async_copy(rows[j], out2.at[pl.ds(0, G)], osem[j]).wait()

    for u0 in range(AHEAD):
        fire_gather(u0, u0 % NBUF)

    def phase(u, j):
        drain_gather(j)
        fire_wb(u, j)
        j2 = (j + AHEAD) % NBUF

        @pl.when(u + AHEAD >= NBUF)
        def _():
            drain_wb(j2)

        @pl.when(u + AHEAD < NU)
        def _():
            fire_gather(u + AHEAD, j2)

    def round_(k, carry):
        for j in range(NBUF):
            phase(NBUF * k + j, j)
        return carry

    lax.fori_loop(0, ROUNDS, round_, 0)
    for u in range(NBUF * ROUNDS, NU):
        j = u % NBUF
        drain_gather(j)
        fire_wb(u, j)
        j2 = (j + AHEAD) % NBUF
        drain_wb(j2)
        if u + AHEAD < NU:
            fire_gather(u + AHEAD, j2)
    for u in range(NU - AHEAD, NU):
        drain_wb(u % NBUF)


@jax.jit
def _emb(xb, tfat):
    mesh = plsc.VectorSubcoreMesh(core_axis_name="c", subcore_axis_name="s")
    run = pl.kernel(
        _emb_body,
        out_type=jax.ShapeDtypeStruct((B, 128), jnp.float32),
        mesh=mesh,
        scratch_types=(
            [pltpu.VMEM((NU, G), jnp.int32)]
            + [pltpu.VMEM((G, 128), jnp.float32)] * NBUF
            + [pltpu.SemaphoreType.DMA] * (2 * NBUF)
        ),
        compiler_params=pltpu.CompilerParams(needs_layout_passes=False),
    )
    return run(xb, tfat)


def kernel(x, table):
    xb = x.reshape(NW, NU, G).astype(jnp.int32)
    tfat = jnp.pad(table, ((0, 0), (0, 128 - DIM)))
    out2 = _emb(xb, tfat)
    return out2[:, :DIM].reshape(BATCH, HIST, DIM)
